# baseline (speedup 1.0000x reference)
import functools

import jax
import jax.numpy as jnp
import numpy as np
from jax.experimental import pallas as pl
from jax.experimental.pallas import tpu as pltpu
from jax.experimental.pallas import tpu_sc as plsc

EMBED = 128
NVS = 4
N_USERS = 500
M_ITEMS = 500
VOCAB = N_USERS + M_ITEMS
BATCH = 16384
V = 1024
CHUNK = 1024
NCHUNK = BATCH // CHUNK
ECOLS = NVS * EMBED


def _compute_consts():
    cpu = jax.devices("cpu")[0]
    with jax.default_device(cpu):
        kr = jax.random.key(42)
        k1, k2, k3 = jax.random.split(kr, 3)
        noise_g = np.asarray(jax.random.normal(k1, (NVS, 1), dtype=jnp.float32))
        noise_b = np.asarray(jax.random.normal(k2, (NVS, VOCAB), dtype=jnp.float32))
        noise_e = np.asarray(jax.random.normal(k3, (NVS, VOCAB, EMBED), dtype=jnp.float32))
    n_mat = np.zeros((V, ECOLS + NVS), dtype=np.float32)
    n_mat[:VOCAB, :ECOLS] = np.transpose(noise_e, (1, 0, 2)).reshape(VOCAB, ECOLS)
    n_mat[:VOCAB, ECOLS:] = noise_b.T
    return n_mat, noise_g


_N_MAT, _NOISE_G = _compute_consts()


def _kl(m, s):
    return -jnp.log(s) + (s * s + m * m) * 0.5 - 0.5


def _stage_a(xu_r, xi_r, bias_w, ent_w, n_mat, gbm, gbs, alpha,
             m_ent, bsum, klt, std):
    f32 = jnp.float32
    ids_row = jax.lax.broadcasted_iota(jnp.int32, (1, V), 1).astype(f32)
    ids_col = jax.lax.broadcasted_iota(jnp.int32, (V, 1), 0).astype(f32)

    def hist_col(ref):
        def body(c, acc):
            xr = ref[pl.ds(c, 1), :].astype(f32)
            eq = (ids_col == xr).astype(f32)
            return acc + jnp.sum(eq, axis=1, keepdims=True)
        return jax.lax.fori_loop(0, NCHUNK, body, jnp.zeros((V, 1), f32))

    occ_u_col = hist_col(xu_r)
    occ_i_col = hist_col(xi_r)
    occ_col = occ_u_col + occ_i_col

    present_col = (occ_col > 0.0).astype(f32)

    r_io = jax.lax.broadcasted_iota(jnp.int32, (V, V), 0).astype(f32)
    c_io = jax.lax.broadcasted_iota(jnp.int32, (V, V), 1).astype(f32)
    tri = (c_io < r_io).astype(f32)

    rank_col = jnp.dot(tri, present_col, preferred_element_type=f32)

    P = (rank_col == ids_row).astype(f32) * present_col

    G = jnp.dot(P, n_mat[...], preferred_element_type=f32,
                precision=jax.lax.Precision.HIGHEST)

    ew = ent_w[...]
    ew = jnp.where(ew != ew, jnp.float32(1e-6), ew)
    ew = jnp.clip(ew, jnp.float32(-3.4028235e38), jnp.float32(3.4028235e38))
    ent_loc = ew[:, :EMBED]
    ent_scale = jnp.abs(ew[:, EMBED:])
    bias_loc = bias_w[:, 0:1]
    bias_scale = jnp.abs(bias_w[:, 1:2])

    ge = G[:, :ECOLS]
    loc4 = jnp.concatenate([ent_loc] * NVS, axis=1)
    sc4 = jnp.concatenate([ent_scale] * NVS, axis=1)
    m_ent[...] = loc4 + sc4 * ge

    gb_sum = jnp.sum(G[:, ECOLS:], axis=1, keepdims=True)
    bsum[...] = jnp.float32(NVS) * bias_loc + bias_scale * gb_sum

    kv = _kl(bias_loc, bias_scale) + jnp.sum(_kl(ent_loc, ent_scale),
                                             axis=1, keepdims=True)

    tdims = (((0,), (0,)), ((), ()))
    cnt_pos = jax.lax.dot_general(P, occ_col, tdims, preferred_element_type=f32,
                                  precision=jax.lax.Precision.HIGHEST)
    uniq_pos = jax.lax.dot_general(P, ids_col, tdims, preferred_element_type=f32,
                                   precision=jax.lax.Precision.HIGHEST)
    Q = (uniq_pos == ids_row).astype(f32)
    r2 = jnp.dot(Q, cnt_pos, preferred_element_type=f32,
                 precision=jax.lax.Precision.HIGHEST)

    user_norm = jnp.sum(jnp.where(occ_u_col > 0.0, occ_u_col / r2, 0.0))
    item_norm = jnp.sum(jnp.where(occ_i_col > 0.0, occ_i_col / r2, 0.0))

    idsc = ids_col
    sel = (jnp.where(idsc <= N_USERS, jnp.float32(N_USERS) / user_norm, 0.0)
           + jnp.where(idsc > N_USERS, jnp.float32(M_ITEMS) / item_norm, 0.0))
    term = jnp.where(occ_col > 0.0, occ_col / cnt_pos, 0.0)
    kl_rescaled = jnp.sum(kv * term * sel)

    gbs_a = jnp.abs(gbs[...])
    kl_global = _kl(gbm[...], gbs_a)
    klt[...] = kl_global + kl_rescaled
    std[...] = jnp.sqrt(1.0 / jnp.abs(alpha[...]))


NW = 32
BPW = BATCH // NW
GROUP = 64
NG = BPW // GROUP


def _sc_stage_b_body(tab_hbm, xu_hbm, xi_hbm, bsum_hbm, s_hbm,
                     idxu_v, idxi_v, bsum_v, ru_v, ri_v, red_v, s_v,
                     semu, semi):
    f32 = jnp.float32
    i32 = jnp.int32
    cid = jax.lax.axis_index("c")
    sid = jax.lax.axis_index("s")
    wid = sid * 2 + cid
    base = wid * BPW
    pltpu.sync_copy(xu_hbm.at[pl.ds(base, BPW)], idxu_v)
    pltpu.sync_copy(xi_hbm.at[pl.ds(base, BPW)], idxi_v)
    pltpu.sync_copy(bsum_hbm, bsum_v)
    lanes16 = jax.lax.iota(i32, 16) * 16
    for g in range(NG):
        cu = pltpu.async_copy(tab_hbm.at[idxu_v.at[pl.ds(g * GROUP, GROUP)]],
                              ru_v, semu)
        ci = pltpu.async_copy(tab_hbm.at[idxi_v.at[pl.ds(g * GROUP, GROUP)]],
                              ri_v, semi)
        cu.wait()
        ci.wait()

        def group16(g2, _):
            goff = g2 * 16

            def elem(le, _2):
                e = goff + le

                def chunk(c, acc):
                    cc = c * 64
                    for k in range(4):
                        u = ru_v[e, pl.ds(cc + k * 16, 16)]
                        w = ri_v[e, pl.ds(cc + k * 16, 16)]
                        acc = acc + u * w
                    return acc

                acc = jax.lax.fori_loop(0, ECOLS // 64, chunk,
                                        jnp.zeros((16,), f32))
                red_v[pl.ds(le * 16, 16)] = acc
                return 0

            jax.lax.fori_loop(0, 16, elem, 0)
            s16 = jnp.zeros((16,), f32)
            for c in range(16):
                s16 = s16 + plsc.load_gather(red_v, [lanes16 + c])
            boff = g * GROUP + goff
            bu = plsc.load_gather(bsum_v, [idxu_v[pl.ds(boff, 16)]])
            bi = plsc.load_gather(bsum_v, [idxi_v[pl.ds(boff, 16)]])
            s_v[pl.ds(boff, 16)] = (s16 + bu + bi) * f32(1.0 / NVS)
            return 0

        jax.lax.fori_loop(0, GROUP // 16, group16, 0)
    pltpu.sync_copy(s_v, s_hbm.at[pl.ds(base, BPW)])


def _sc_stage_b(m_ent, xu, xi, bsum):
    f32 = jnp.float32
    mesh = plsc.VectorSubcoreMesh(core_axis_name="c", subcore_axis_name="s")
    k = functools.partial(
        pl.kernel,
        out_type=jax.ShapeDtypeStruct((BATCH,), f32),
        mesh=mesh,
        compiler_params=pltpu.CompilerParams(needs_layout_passes=False),
        scratch_types=[
            pltpu.VMEM((BPW,), jnp.int32),
            pltpu.VMEM((BPW,), jnp.int32),
            pltpu.VMEM((V,), f32),
            pltpu.VMEM((GROUP, ECOLS), f32),
            pltpu.VMEM((GROUP, ECOLS), f32),
            pltpu.VMEM((256,), f32),
            pltpu.VMEM((BPW,), f32),
            pltpu.SemaphoreType.DMA,
            pltpu.SemaphoreType.DMA,
        ],
    )(_sc_stage_b_body)
    return k(m_ent, xu, xi, bsum)


@jax.jit
def _run(x, alpha, gbm, gbs, bias_weight, entity_weight, n_mat, noise_g):
    f32 = jnp.float32
    xu = x[:, 0]
    xi = x[:, 1]
    xu_r = xu.reshape(NCHUNK, CHUNK)
    xi_r = xi.reshape(NCHUNK, CHUNK)
    bias_pad = jnp.concatenate(
        [bias_weight, jnp.tile(jnp.array([[0.0, 1.0]], f32), (V - VOCAB, 1))], axis=0)
    ent_pad = jnp.concatenate(
        [entity_weight,
         jnp.concatenate([jnp.zeros((V - VOCAB, EMBED), f32),
                          jnp.ones((V - VOCAB, EMBED), f32)], axis=1)], axis=0)

    m_ent, bsum, klt, std = pl.pallas_call(
        _stage_a,
        out_shape=[
            jax.ShapeDtypeStruct((V, ECOLS), f32),
            jax.ShapeDtypeStruct((V, 1), f32),
            jax.ShapeDtypeStruct((1, 1), f32),
            jax.ShapeDtypeStruct((1, 1), f32),
        ],
    )(xu_r, xi_r, bias_pad, ent_pad,
      n_mat, gbm.reshape(1, 1), gbs.reshape(1, 1), alpha.reshape(1, 1))

    s = _sc_stage_b(m_ent, xu, xi, bsum.reshape(V))

    gb = gbm + jnp.abs(gbs) * noise_g
    pred = gb + s.reshape(1, BATCH)
    return pred, std.reshape(1), klt.reshape(1)


def kernel(x, alpha, global_bias_mean, global_bias_scale, bias_weight, entity_weight):
    return _run(x, alpha, global_bias_mean, global_bias_scale,
                bias_weight, entity_weight, jnp.asarray(_N_MAT), jnp.asarray(_NOISE_G))


# speedup vs baseline: 1.0714x; 1.0714x over previous
import functools

import jax
import jax.numpy as jnp
import numpy as np
from jax.experimental import pallas as pl
from jax.experimental.pallas import tpu as pltpu
from jax.experimental.pallas import tpu_sc as plsc

EMBED = 128
NVS = 4
N_USERS = 500
M_ITEMS = 500
VOCAB = N_USERS + M_ITEMS
BATCH = 16384
V = 1024
CHUNK = 1024
NCHUNK = BATCH // CHUNK
ECOLS = NVS * EMBED


def _compute_consts():
    cpu = jax.devices("cpu")[0]
    with jax.default_device(cpu):
        kr = jax.random.key(42)
        k1, k2, k3 = jax.random.split(kr, 3)
        noise_g = np.asarray(jax.random.normal(k1, (NVS, 1), dtype=jnp.float32))
        noise_b = np.asarray(jax.random.normal(k2, (NVS, VOCAB), dtype=jnp.float32))
        noise_e = np.asarray(jax.random.normal(k3, (NVS, VOCAB, EMBED), dtype=jnp.float32))
    n_mat = np.zeros((V, ECOLS + NVS), dtype=np.float32)
    n_mat[:VOCAB, :ECOLS] = np.transpose(noise_e, (1, 0, 2)).reshape(VOCAB, ECOLS)
    n_mat[:VOCAB, ECOLS:] = noise_b.T
    return n_mat, noise_g


_N_MAT, _NOISE_G = _compute_consts()


def _kl(m, s):
    return -jnp.log(s) + (s * s + m * m) * 0.5 - 0.5


def _stage_a(xu_c, xi_c, bias_w, ent_w, n_mat, gbm, gbs, alpha,
             m_ent, bsum, klt, std):
    f32 = jnp.float32
    i32 = jnp.int32
    ids_row = jax.lax.broadcasted_iota(i32, (1, V), 1).astype(f32)
    ids_col = jax.lax.broadcasted_iota(i32, (V, 1), 0).astype(f32)
    tdims = (((0,), (0,)), ((), ()))

    hi32_row = jax.lax.broadcasted_iota(i32, (1, 32), 1)
    idsv_col = jax.lax.broadcasted_iota(i32, (V, 1), 0)
    ihi = (jax.lax.shift_right_logical(idsv_col, 5) == hi32_row).astype(f32)
    ilo = ((idsv_col & 31) == hi32_row).astype(f32)

    def hist(ref):
        xv = ref[...]
        ehi = (jax.lax.shift_right_logical(xv, 5) == hi32_row).astype(f32)
        elo = ((xv & 31) == hi32_row).astype(f32)
        omat = jax.lax.dot_general(ehi, elo, tdims, preferred_element_type=f32)
        a = jnp.dot(ihi, omat, preferred_element_type=f32,
                    precision=jax.lax.Precision.HIGHEST)
        return jnp.sum(a * ilo, axis=1, keepdims=True)

    occ_u_col = hist(xu_c)
    occ_i_col = hist(xi_c)
    occ_col = occ_u_col + occ_i_col

    present_col = (occ_col > 0.0).astype(f32)

    r_io = jax.lax.broadcasted_iota(jnp.int32, (V, V), 0).astype(f32)
    c_io = jax.lax.broadcasted_iota(jnp.int32, (V, V), 1).astype(f32)
    tri = (c_io < r_io).astype(f32)

    rank_col = jnp.dot(tri, present_col, preferred_element_type=f32)

    P = (rank_col == ids_row).astype(f32) * present_col

    G = jnp.dot(P, n_mat[...], preferred_element_type=f32,
                precision=jax.lax.Precision.HIGHEST)

    ew = ent_w[...]
    ew = jnp.where(ew != ew, jnp.float32(1e-6), ew)
    ew = jnp.clip(ew, jnp.float32(-3.4028235e38), jnp.float32(3.4028235e38))
    ent_loc = ew[:, :EMBED]
    ent_scale = jnp.abs(ew[:, EMBED:])
    bias_loc = bias_w[:, 0:1]
    bias_scale = jnp.abs(bias_w[:, 1:2])

    ge = G[:, :ECOLS]
    loc4 = jnp.concatenate([ent_loc] * NVS, axis=1)
    sc4 = jnp.concatenate([ent_scale] * NVS, axis=1)
    m_ent[...] = loc4 + sc4 * ge

    gb_sum = jnp.sum(G[:, ECOLS:], axis=1, keepdims=True)
    bsum[...] = jnp.float32(NVS) * bias_loc + bias_scale * gb_sum

    kv = _kl(bias_loc, bias_scale) + jnp.sum(_kl(ent_loc, ent_scale),
                                             axis=1, keepdims=True)

    cnt_pos = jax.lax.dot_general(P, occ_col, tdims, preferred_element_type=f32,
                                  precision=jax.lax.Precision.HIGHEST)
    uniq_pos = jax.lax.dot_general(P, ids_col, tdims, preferred_element_type=f32,
                                   precision=jax.lax.Precision.HIGHEST)
    Q = (uniq_pos == ids_row).astype(f32)
    r2 = jnp.dot(Q, cnt_pos, preferred_element_type=f32,
                 precision=jax.lax.Precision.HIGHEST)

    user_norm = jnp.sum(jnp.where(occ_u_col > 0.0, occ_u_col / r2, 0.0))
    item_norm = jnp.sum(jnp.where(occ_i_col > 0.0, occ_i_col / r2, 0.0))

    idsc = ids_col
    sel = (jnp.where(idsc <= N_USERS, jnp.float32(N_USERS) / user_norm, 0.0)
           + jnp.where(idsc > N_USERS, jnp.float32(M_ITEMS) / item_norm, 0.0))
    term = jnp.where(occ_col > 0.0, occ_col / cnt_pos, 0.0)
    kl_rescaled = jnp.sum(kv * term * sel)

    gbs_a = jnp.abs(gbs[...])
    kl_global = _kl(gbm[...], gbs_a)
    klt[...] = kl_global + kl_rescaled
    std[...] = jnp.sqrt(1.0 / jnp.abs(alpha[...]))


NW = 32
BPW = BATCH // NW
GROUP = 32
NG = BPW // GROUP


def _sc_stage_b_body(tab_hbm, xu_hbm, xi_hbm, bsum_hbm, s_hbm,
                     idxu_v, idxi_v, bsum_v,
                     ru0, ri0, ru1, ri1, red_v, s_v,
                     semu0, semi0, semu1, semi1):
    f32 = jnp.float32
    i32 = jnp.int32
    cid = jax.lax.axis_index("c")
    sid = jax.lax.axis_index("s")
    wid = sid * 2 + cid
    base = wid * BPW
    pltpu.sync_copy(xu_hbm.at[pl.ds(base, BPW)], idxu_v)
    pltpu.sync_copy(xi_hbm.at[pl.ds(base, BPW)], idxi_v)
    pltpu.sync_copy(bsum_hbm, bsum_v)
    lanes16 = jax.lax.iota(i32, 16) * 16
    bufs = [(ru0, ri0, semu0, semi0), (ru1, ri1, semu1, semi1)]

    def fire(g):
        ru, ri, su, si = bufs[g % 2]
        cu = pltpu.async_copy(tab_hbm.at[idxu_v.at[pl.ds(g * GROUP, GROUP)]],
                              ru, su)
        ci = pltpu.async_copy(tab_hbm.at[idxi_v.at[pl.ds(g * GROUP, GROUP)]],
                              ri, si)
        return cu, ci

    pend = fire(0)
    for g in range(NG):
        cu, ci = pend
        cu.wait()
        ci.wait()
        if g + 1 < NG:
            pend = fire(g + 1)
        ru, ri, _, _ = bufs[g % 2]

        def group16(g2, _, ru=ru, ri=ri, g=g):
            goff = g2 * 16

            def elem(le, _2):
                e = goff + le
                acc = jnp.zeros((16,), f32)
                for c in range(ECOLS // 16):
                    u = ru[e, pl.ds(c * 16, 16)]
                    w = ri[e, pl.ds(c * 16, 16)]
                    acc = acc + u * w
                red_v[pl.ds(le * 16, 16)] = acc
                return 0

            jax.lax.fori_loop(0, 16, elem, 0)
            s16 = jnp.zeros((16,), f32)
            for c in range(16):
                s16 = s16 + plsc.load_gather(red_v, [lanes16 + c])
            boff = g * GROUP + goff
            bu = plsc.load_gather(bsum_v, [idxu_v[pl.ds(boff, 16)]])
            bi = plsc.load_gather(bsum_v, [idxi_v[pl.ds(boff, 16)]])
            s_v[pl.ds(boff, 16)] = (s16 + bu + bi) * f32(1.0 / NVS)
            return 0

        jax.lax.fori_loop(0, GROUP // 16, group16, 0)
    pltpu.sync_copy(s_v, s_hbm.at[pl.ds(base, BPW)])


def _sc_stage_b(m_ent, xu, xi, bsum):
    f32 = jnp.float32
    mesh = plsc.VectorSubcoreMesh(core_axis_name="c", subcore_axis_name="s")
    k = functools.partial(
        pl.kernel,
        out_type=jax.ShapeDtypeStruct((BATCH,), f32),
        mesh=mesh,
        compiler_params=pltpu.CompilerParams(needs_layout_passes=False),
        scratch_types=[
            pltpu.VMEM((BPW,), jnp.int32),
            pltpu.VMEM((BPW,), jnp.int32),
            pltpu.VMEM((V,), f32),
            pltpu.VMEM((GROUP, ECOLS), f32),
            pltpu.VMEM((GROUP, ECOLS), f32),
            pltpu.VMEM((GROUP, ECOLS), f32),
            pltpu.VMEM((GROUP, ECOLS), f32),
            pltpu.VMEM((256,), f32),
            pltpu.VMEM((BPW,), f32),
            pltpu.SemaphoreType.DMA,
            pltpu.SemaphoreType.DMA,
            pltpu.SemaphoreType.DMA,
            pltpu.SemaphoreType.DMA,
        ],
    )(_sc_stage_b_body)
    return k(m_ent, xu, xi, bsum)


@jax.jit
def _run(x, alpha, gbm, gbs, bias_weight, entity_weight, n_mat, noise_g):
    f32 = jnp.float32
    xu = x[:, 0]
    xi = x[:, 1]
    bias_pad = jnp.concatenate(
        [bias_weight, jnp.tile(jnp.array([[0.0, 1.0]], f32), (V - VOCAB, 1))], axis=0)
    ent_pad = jnp.concatenate(
        [entity_weight,
         jnp.concatenate([jnp.zeros((V - VOCAB, EMBED), f32),
                          jnp.ones((V - VOCAB, EMBED), f32)], axis=1)], axis=0)

    m_ent, bsum, klt, std = pl.pallas_call(
        _stage_a,
        out_shape=[
            jax.ShapeDtypeStruct((V, ECOLS), f32),
            jax.ShapeDtypeStruct((V, 1), f32),
            jax.ShapeDtypeStruct((1, 1), f32),
            jax.ShapeDtypeStruct((1, 1), f32),
        ],
    )(xu.reshape(BATCH, 1), xi.reshape(BATCH, 1), bias_pad, ent_pad,
      n_mat, gbm.reshape(1, 1), gbs.reshape(1, 1), alpha.reshape(1, 1))

    s = _sc_stage_b(m_ent, xu, xi, bsum.reshape(V))

    gb = gbm + jnp.abs(gbs) * noise_g
    pred = gb + s.reshape(1, BATCH)
    return pred, std.reshape(1), klt.reshape(1)


def kernel(x, alpha, global_bias_mean, global_bias_scale, bias_weight, entity_weight):
    return _run(x, alpha, global_bias_mean, global_bias_scale,
                bias_weight, entity_weight, jnp.asarray(_N_MAT), jnp.asarray(_NOISE_G))


# speedup vs baseline: 1.1529x; 1.0761x over previous
import functools

import jax
import jax.numpy as jnp
import numpy as np
from jax import lax
from jax.experimental import pallas as pl
from jax.experimental.pallas import tpu as pltpu
from jax.experimental.pallas import tpu_sc as plsc

EMBED = 128
NVS = 4
N_USERS = 500
M_ITEMS = 500
VOCAB = N_USERS + M_ITEMS
BATCH = 16384
V = 1024
ECOLS = NVS * EMBED
NPC = 640
TABW = ECOLS // 2

NW = 32
BPW = BATCH // NW
GROUP = 32
NG = BPW // GROUP
RPT = V // 16


def _compute_consts():
    cpu = jax.devices("cpu")[0]
    with jax.default_device(cpu):
        kr = jax.random.key(42)
        k1, k2, k3 = jax.random.split(kr, 3)
        noise_g = np.asarray(jax.random.normal(k1, (NVS, 1), dtype=jnp.float32))
        noise_b = np.asarray(jax.random.normal(k2, (NVS, VOCAB), dtype=jnp.float32))
        noise_e = np.asarray(jax.random.normal(k3, (NVS, VOCAB, EMBED), dtype=jnp.float32))
    n_pad = np.zeros((V, NPC), dtype=np.float32)
    n_pad[:VOCAB, :ECOLS] = np.transpose(noise_e, (1, 0, 2)).reshape(VOCAB, ECOLS)
    n_pad[:VOCAB, ECOLS:ECOLS + NVS] = noise_b.T
    return n_pad, noise_g


_N_PAD, _NOISE_G = _compute_consts()


def _kl(m, s):
    return -jnp.log(s) + (s * s + m * m) * 0.5 - 0.5


def _stage_a(xu_c, xi_c, bias_w, ent_w, gbm, gbs, alpha,
             occu_o, occi_o, occ_o, kv_o, entp_o, bloc_o, bsc_o, klg_o, std_o):
    f32 = jnp.float32
    i32 = jnp.int32
    tdims = (((0,), (0,)), ((), ()))

    hi32_row = jax.lax.broadcasted_iota(i32, (1, 32), 1)
    idsv_col = jax.lax.broadcasted_iota(i32, (V, 1), 0)
    ihi = (jax.lax.shift_right_logical(idsv_col, 5) == hi32_row).astype(f32)
    ilo = ((idsv_col & 31) == hi32_row).astype(f32)

    def hist(ref):
        xv = ref[...]
        ehi = (jax.lax.shift_right_logical(xv, 5) == hi32_row).astype(f32)
        elo = ((xv & 31) == hi32_row).astype(f32)
        omat = jax.lax.dot_general(ehi, elo, tdims, preferred_element_type=f32)
        a = jnp.dot(ihi, omat, preferred_element_type=f32,
                    precision=jax.lax.Precision.HIGHEST)
        return jnp.sum(a * ilo, axis=1, keepdims=True)

    occ_u = hist(xu_c)
    occ_i = hist(xi_c)
    occu_o[...] = occ_u
    occi_o[...] = occ_i
    occ_o[...] = occ_u + occ_i

    ew = ent_w[...]
    ew = jnp.where(ew != ew, jnp.float32(1e-6), ew)
    ew = jnp.clip(ew, jnp.float32(-3.4028235e38), jnp.float32(3.4028235e38))
    ent_loc = ew[:, :EMBED]
    ent_scale = jnp.abs(ew[:, EMBED:])
    entp_o[...] = jnp.concatenate([ent_loc, ent_scale], axis=1)
    bias_loc = bias_w[:, 0:1]
    bias_scale = jnp.abs(bias_w[:, 1:2])
    bloc_o[...] = bias_loc
    bsc_o[...] = bias_scale

    kv_o[...] = _kl(bias_loc, bias_scale) + jnp.sum(_kl(ent_loc, ent_scale),
                                                    axis=1, keepdims=True)

    klg_o[...] = _kl(gbm[...], jnp.abs(gbs[...]))
    std_o[...] = jnp.sqrt(1.0 / jnp.abs(alpha[...]))


def _rcp(v):
    x = plsc.bitcast(jnp.int32(0x7EF311C3) - plsc.bitcast(v, jnp.int32),
                     jnp.float32)
    for _ in range(3):
        x = x * (2.0 - v * x)
    return x


def _lanesum(vec):
    s = vec[0]
    for j in range(1, 16):
        s = s + vec[j]
    return s


def _sc_main_body(n_hbm, entp_hbm, bloc_hbm, bsc_hbm, occu_hbm, occi_hbm, occ_hbm,
                  kv_hbm, xu_hbm, xi_hbm,
                  s_hbm, krs_hbm, tab0_hbm, tab1_hbm,
                  sh_rank, sh_bsum,
                  occ_v, occu_v, occi_v, kv_v, rank_v, cnt_v, uniq_v, r2_v,
                  kbuf, myrank_v, n_blk, ent_blk, bloc_blk, bsc_blk, mrow_v,
                  bs64_v, idxu_v, idxi_v, bsum_v, ru0, ri0, ru1, ri1,
                  red_v, s_v,
                  sem, semu0, semi0, semu1, semi1):
    f32 = jnp.float32
    i32 = jnp.int32
    cid = lax.axis_index("c")
    sid = lax.axis_index("s")
    lanes = lax.iota(i32, 16)

    @pl.when(sid == 0)
    def _():
        pltpu.sync_copy(occ_hbm, occ_v)

        def rbody(c, carry):
            oc = occ_v[pl.ds(c * 16, 16)]
            pr = jnp.where(oc > 0.0, 1.0, 0.0).astype(f32)
            cs = plsc.cumsum(pr)
            rank_v[pl.ds(c * 16, 16)] = (cs - pr + carry).astype(i32)
            return carry + cs[15]

        lax.fori_loop(0, V // 16, rbody, jnp.float32(0.0))
        pltpu.sync_copy(rank_v, sh_rank)

    plsc.subcore_barrier()

    r0 = sid * RPT
    pltpu.sync_copy(sh_rank.at[pl.ds(r0, RPT)], myrank_v)
    pltpu.async_copy(n_hbm.at[myrank_v], n_blk, sem).wait()
    pltpu.sync_copy(entp_hbm.at[pl.ds(r0, RPT)], ent_blk)
    pltpu.sync_copy(bloc_hbm.at[pl.ds(r0, RPT)], bloc_blk)
    pltpu.sync_copy(bsc_hbm.at[pl.ds(r0, RPT)], bsc_blk)
    bmask = jnp.where(lanes < NVS, 1.0, 0.0).astype(f32)

    def rowbody(r, _):
        for c in range(16):
            ca = 32 * c
            cb = 32 * c + 16
            a = (ent_blk[r, pl.ds(ca % 128, 16)]
                 + ent_blk[r, pl.ds(128 + ca % 128, 16)] * n_blk[r, pl.ds(ca, 16)])
            b = (ent_blk[r, pl.ds(cb % 128, 16)]
                 + ent_blk[r, pl.ds(128 + cb % 128, 16)] * n_blk[r, pl.ds(cb, 16)])
            w = plsc.bitcast(
                plsc.pack(a, b, format=plsc.PackFormat.INTERLEAVED), i32)
            mrow_v[r, pl.ds(16 * c, 16)] = w
        return 0

    lax.fori_loop(0, RPT, rowbody, 0)

    def quadbody(q, _):
        lc16 = bloc_blk[pl.ds(q * 16, 16)]
        sc16 = bsc_blk[pl.ds(q * 16, 16)]
        bs16 = jnp.zeros((16,), f32)
        for rl in range(16):
            nb = n_blk[q * 16 + rl, pl.ds(ECOLS, 16)] * bmask
            sb = plsc.cumsum(nb)[15]
            bs16 = bs16 + jnp.where(lanes == rl, jnp.full((16,), sb, f32), 0.0)
        bs64_v[pl.ds(q * 16, 16)] = 4.0 * lc16 + sc16 * bs16
        return 0

    lax.fori_loop(0, RPT // 16, quadbody, 0)

    @pl.when(cid == 0)
    def _():
        pltpu.sync_copy(mrow_v, tab0_hbm.at[pl.ds(r0, RPT)])

    @pl.when(cid == 1)
    def _():
        pltpu.sync_copy(mrow_v, tab1_hbm.at[pl.ds(r0, RPT)])

    pltpu.sync_copy(bs64_v, sh_bsum.at[pl.ds(r0, RPT)])

    @pl.when(sid == 0)
    def _():
        pltpu.sync_copy(occu_hbm, occu_v)
        pltpu.sync_copy(occi_hbm, occi_v)
        pltpu.sync_copy(kv_hbm, kv_v)
        zero16 = jnp.zeros((16,), f32)

        def zbody(c, _2):
            cnt_v[pl.ds(c * 16, 16)] = zero16
            uniq_v[pl.ds(c * 16, 16)] = zero16
            return 0

        lax.fori_loop(0, V // 16, zbody, 0)

        def sbody(c, _2):
            oc = occ_v[pl.ds(c * 16, 16)]
            rk = rank_v[pl.ds(c * 16, 16)]
            pr = oc > 0.0
            plsc.store_scatter(cnt_v, [rk], oc, mask=pr)
            ids16 = (lanes + c * 16).astype(f32)
            plsc.store_scatter(uniq_v, [rk], ids16, mask=pr)
            return 0

        lax.fori_loop(0, V // 16, sbody, 0)

        def gbody(c, _2):
            up = uniq_v[pl.ds(c * 16, 16)].astype(i32)
            r2_v[pl.ds(c * 16, 16)] = plsc.load_gather(cnt_v, [up])
            return 0

        lax.fori_loop(0, V // 16, gbody, 0)

        def nbody(c, accs):
            au, ai = accs
            r2c = r2_v[pl.ds(c * 16, 16)]
            ou = occu_v[pl.ds(c * 16, 16)]
            oi = occi_v[pl.ds(c * 16, 16)]
            rr = _rcp(r2c)
            au = au + jnp.where(ou > 0.0, ou * rr, 0.0)
            ai = ai + jnp.where(oi > 0.0, oi * rr, 0.0)
            return (au, ai)

        au, ai = lax.fori_loop(0, V // 16, nbody, (jnp.zeros((16,), f32),
                                                   jnp.zeros((16,), f32)))
        run16 = _rcp(jnp.full((16,), _lanesum(au), f32)) * jnp.float32(N_USERS)
        rin16 = _rcp(jnp.full((16,), _lanesum(ai), f32)) * jnp.float32(M_ITEMS)

        def kbody(c, acc):
            oc = occ_v[pl.ds(c * 16, 16)]
            cp = cnt_v[pl.ds(c * 16, 16)]
            kvc = kv_v[pl.ds(c * 16, 16)]
            ids16 = (lanes + c * 16).astype(f32)
            sel = (jnp.where(ids16 <= jnp.float32(N_USERS), run16, 0.0)
                   + jnp.where(ids16 > jnp.float32(N_USERS), rin16, 0.0))
            term = jnp.where(oc > 0.0, oc * _rcp(cp), 0.0)
            return acc + kvc * term * sel

        kacc = lax.fori_loop(0, V // 16, kbody, jnp.zeros((16,), f32))
        krs = _lanesum(kacc)
        kbuf[...] = jnp.where(lanes == 0, jnp.full((16,), krs, f32), 0.0)

        @pl.when(cid == 0)
        def _():
            pltpu.sync_copy(kbuf, krs_hbm)

    plsc.subcore_barrier()

    pltpu.sync_copy(sh_bsum, bsum_v)
    wid = sid * 2 + cid
    base = wid * BPW
    pltpu.sync_copy(xu_hbm.at[pl.ds(base, BPW)], idxu_v)
    pltpu.sync_copy(xi_hbm.at[pl.ds(base, BPW)], idxi_v)
    lanes16 = lanes * 16

    def phase2(tab):
        def fire(g, ru, ri, su, si):
            cu = pltpu.async_copy(tab.at[idxu_v.at[pl.ds(g * GROUP, GROUP)]],
                                  ru, su)
            ci = pltpu.async_copy(tab.at[idxi_v.at[pl.ds(g * GROUP, GROUP)]],
                                  ri, si)
            return cu, ci

        def wait(g, ru, ri, su, si):
            pltpu.make_async_copy(tab.at[idxu_v.at[pl.ds(g * GROUP, GROUP)]],
                                  ru, su).wait()
            pltpu.make_async_copy(tab.at[idxi_v.at[pl.ds(g * GROUP, GROUP)]],
                                  ri, si).wait()

        def compute(g, ru, ri):
            def group16(g2, _):
                goff = g2 * 16

                def elem(le, _2):
                    e = goff + le
                    acc = jnp.zeros((16,), f32)
                    for c in range(TABW // 16):
                        ui = ru[e, pl.ds(c * 16, 16)]
                        wi = ri[e, pl.ds(c * 16, 16)]
                        u32 = plsc.bitcast(ui, jnp.bfloat16)
                        w32 = plsc.bitcast(wi, jnp.bfloat16)
                        ua, ub = plsc.unpack(
                            u32, format=plsc.PackFormat.INTERLEAVED,
                            preferred_element_type=f32)
                        wa, wb = plsc.unpack(
                            w32, format=plsc.PackFormat.INTERLEAVED,
                            preferred_element_type=f32)
                        acc = acc + ua * wa + ub * wb
                    red_v[pl.ds(le * 16, 16)] = acc
                    return 0

                lax.fori_loop(0, 16, elem, 0)
                s16 = jnp.zeros((16,), f32)
                for c in range(16):
                    s16 = s16 + plsc.load_gather(red_v, [lanes16 + c])
                boff = g * GROUP + goff
                bu = plsc.load_gather(bsum_v, [idxu_v[pl.ds(boff, 16)]])
                bi = plsc.load_gather(bsum_v, [idxi_v[pl.ds(boff, 16)]])
                s_v[pl.ds(boff, 16)] = (s16 + bu + bi) * f32(1.0 / NVS)
                return 0

            lax.fori_loop(0, GROUP // 16, group16, 0)

        fire(0, ru0, ri0, semu0, semi0)

        def hbody(h, _):
            g0 = 2 * h
            wait(g0, ru0, ri0, semu0, semi0)
            fire(g0 + 1, ru1, ri1, semu1, semi1)
            compute(g0, ru0, ri0)
            wait(g0 + 1, ru1, ri1, semu1, semi1)

            @pl.when(g0 + 2 < NG)
            def _():
                fire(g0 + 2, ru0, ri0, semu0, semi0)

            compute(g0 + 1, ru1, ri1)
            return 0

        lax.fori_loop(0, NG // 2, hbody, 0)

    @pl.when(cid == 0)
    def _():
        phase2(tab0_hbm)

    @pl.when(cid == 1)
    def _():
        phase2(tab1_hbm)

    pltpu.sync_copy(s_v, s_hbm.at[pl.ds(base, BPW)])


def _sc_main(n_pad, entp, bloc, bsc, occu, occi, occ, kv, xu, xi):
    f32 = jnp.float32
    i32 = jnp.int32
    mesh = plsc.VectorSubcoreMesh(core_axis_name="c", subcore_axis_name="s")
    k = functools.partial(
        pl.kernel,
        out_type=[
            jax.ShapeDtypeStruct((BATCH,), f32),
            jax.ShapeDtypeStruct((16,), f32),
            jax.ShapeDtypeStruct((V, TABW), i32),
            jax.ShapeDtypeStruct((V, TABW), i32),
        ],
        mesh=mesh,
        compiler_params=pltpu.CompilerParams(needs_layout_passes=False),
        scratch_types=[
            pltpu.VMEM_SHARED((V,), i32),
            pltpu.VMEM_SHARED((V,), f32),
            pltpu.VMEM((V,), f32),
            pltpu.VMEM((V,), f32),
            pltpu.VMEM((V,), f32),
            pltpu.VMEM((V,), f32),
            pltpu.VMEM((V,), i32),
            pltpu.VMEM((V,), f32),
            pltpu.VMEM((V,), f32),
            pltpu.VMEM((V,), f32),
            pltpu.VMEM((16,), f32),
            pltpu.VMEM((RPT,), i32),
            pltpu.VMEM((RPT, NPC), f32),
            pltpu.VMEM((RPT, 256), f32),
            pltpu.VMEM((RPT,), f32),
            pltpu.VMEM((RPT,), f32),
            pltpu.VMEM((RPT, TABW), i32),
            pltpu.VMEM((RPT,), f32),
            pltpu.VMEM((BPW,), i32),
            pltpu.VMEM((BPW,), i32),
            pltpu.VMEM((V,), f32),
            pltpu.VMEM((GROUP, TABW), i32),
            pltpu.VMEM((GROUP, TABW), i32),
            pltpu.VMEM((GROUP, TABW), i32),
            pltpu.VMEM((GROUP, TABW), i32),
            pltpu.VMEM((256,), f32),
            pltpu.VMEM((BPW,), f32),
            pltpu.SemaphoreType.DMA,
            pltpu.SemaphoreType.DMA,
            pltpu.SemaphoreType.DMA,
            pltpu.SemaphoreType.DMA,
            pltpu.SemaphoreType.DMA,
        ],
    )(_sc_main_body)
    return k(n_pad, entp, bloc, bsc, occu, occi, occ, kv, xu, xi)


@jax.jit
def _run(x, alpha, gbm, gbs, bias_weight, entity_weight, n_pad, noise_g):
    f32 = jnp.float32
    xu = x[:, 0]
    xi = x[:, 1]
    bias_pad = jnp.concatenate(
        [bias_weight, jnp.tile(jnp.array([[0.0, 1.0]], f32), (V - VOCAB, 1))], axis=0)
    ent_pad = jnp.concatenate(
        [entity_weight,
         jnp.concatenate([jnp.zeros((V - VOCAB, EMBED), f32),
                          jnp.ones((V - VOCAB, EMBED), f32)], axis=1)], axis=0)

    occu, occi, occ, kv, entp, bloc, bsc, klg, std = pl.pallas_call(
        _stage_a,
        out_shape=[
            jax.ShapeDtypeStruct((V, 1), f32),
            jax.ShapeDtypeStruct((V, 1), f32),
            jax.ShapeDtypeStruct((V, 1), f32),
            jax.ShapeDtypeStruct((V, 1), f32),
            jax.ShapeDtypeStruct((V, 2 * EMBED), f32),
            jax.ShapeDtypeStruct((V, 1), f32),
            jax.ShapeDtypeStruct((V, 1), f32),
            jax.ShapeDtypeStruct((1, 1), f32),
            jax.ShapeDtypeStruct((1, 1), f32),
        ],
    )(xu.reshape(BATCH, 1), xi.reshape(BATCH, 1), bias_pad, ent_pad,
      gbm.reshape(1, 1), gbs.reshape(1, 1), alpha.reshape(1, 1))

    s, krs, _, _ = _sc_main(n_pad, entp, bloc.reshape(V), bsc.reshape(V),
                            occu.reshape(V), occi.reshape(V), occ.reshape(V),
                            kv.reshape(V), xu, xi)

    gb = gbm + jnp.abs(gbs) * noise_g
    pred = gb + s.reshape(1, BATCH)
    klt = klg.reshape(1) + krs[0:1]
    return pred, std.reshape(1), klt


def kernel(x, alpha, global_bias_mean, global_bias_scale, bias_weight, entity_weight):
    return _run(x, alpha, global_bias_mean, global_bias_scale,
                bias_weight, entity_weight, jnp.asarray(_N_PAD),
                jnp.asarray(_NOISE_G))


# speedup vs baseline: 1.5608x; 1.3537x over previous
import functools

import jax
import jax.numpy as jnp
import numpy as np
from jax import lax
from jax.experimental import pallas as pl
from jax.experimental.pallas import tpu as pltpu
from jax.experimental.pallas import tpu_sc as plsc

EMBED = 128
NVS = 4
N_USERS = 500
M_ITEMS = 500
VOCAB = N_USERS + M_ITEMS
BATCH = 16384
V = 1024
ECOLS = NVS * EMBED
NPC = 640
TABW = ECOLS // 2

NW = 32
BPW = BATCH // NW
GROUP = 32
NG = BPW // GROUP
RPT = V // 16


def _compute_consts():
    cpu = jax.devices("cpu")[0]
    with jax.default_device(cpu):
        kr = jax.random.key(42)
        k1, k2, k3 = jax.random.split(kr, 3)
        noise_g = np.asarray(jax.random.normal(k1, (NVS, 1), dtype=jnp.float32))
        noise_b = np.asarray(jax.random.normal(k2, (NVS, VOCAB), dtype=jnp.float32))
        noise_e = np.asarray(jax.random.normal(k3, (NVS, VOCAB, EMBED), dtype=jnp.float32))
    n_pad = np.zeros((V, NPC), dtype=np.float32)
    n_pad[:VOCAB, :ECOLS] = np.transpose(noise_e, (1, 0, 2)).reshape(VOCAB, ECOLS)
    n_pad[:VOCAB, ECOLS:ECOLS + NVS] = noise_b.T
    return n_pad, noise_g


_N_PAD, _NOISE_G = _compute_consts()


def _kl(m, s):
    return -jnp.log(s) + (s * s + m * m) * 0.5 - 0.5


def _stage_a(bias_w, ent_w, gbm, gbs, alpha, pk_o, entp_o, klg_o, std_o):
    f32 = jnp.float32

    ew = ent_w[...]
    ew = jnp.where(ew != ew, jnp.float32(1e-6), ew)
    ew = jnp.clip(ew, jnp.float32(-3.4028235e38), jnp.float32(3.4028235e38))
    ent_loc = ew[:, :EMBED]
    ent_scale = jnp.abs(ew[:, EMBED:])
    entp_o[...] = jnp.concatenate([ent_loc, ent_scale], axis=1)
    bias_loc = bias_w[:, 0:1]
    bias_scale = jnp.abs(bias_w[:, 1:2])
    kv = _kl(bias_loc, bias_scale) + jnp.sum(_kl(ent_loc, ent_scale),
                                             axis=1, keepdims=True)
    pk_o[...] = jnp.concatenate(
        [kv, bias_loc, bias_scale, jnp.zeros((V, 5), f32)], axis=1)

    klg_o[...] = _kl(gbm[...], jnp.abs(gbs[...]))
    std_o[...] = jnp.sqrt(1.0 / jnp.abs(alpha[...]))


def _rcp(v):
    x = plsc.bitcast(jnp.int32(0x7EF311C3) - plsc.bitcast(v, jnp.int32),
                     jnp.float32)
    for _ in range(3):
        x = x * (2.0 - v * x)
    return x


def _lanesum(vec):
    s = vec[0]
    for j in range(1, 16):
        s = s + vec[j]
    return s


def _sc_main_body(n_hbm, entp_hbm, pk_hbm, xu_hbm, xi_hbm,
                  s_hbm, krs_hbm, tab0_hbm, tab1_hbm,
                  sh_rank, sh_bsum, sh_hist,
                  hu_v, zidx_v, hxu_v, hxi_v, occu_v, occi_v, occ_v, rank_v,
                  cnt_v, uniq_v, r2_v, kbuf, pkkl_v, myrank_v, n_blk,
                  ent_blk, pkb_v, mrow_v, bs64_v, idxu_v, idxi_v, bsum_v,
                  ru0, ri0, ru1, ri1, red_v, s_v,
                  sem, semu0, semi0, semu1, semi1):
    f32 = jnp.float32
    i32 = jnp.int32
    cid = lax.axis_index("c")
    sid = lax.axis_index("s")
    lanes = lax.iota(i32, 16)
    zero16 = jnp.zeros((16,), f32)
    ones16 = jnp.ones((16,), f32)

    hb = sid * (BATCH // 16)
    pltpu.sync_copy(xu_hbm.at[pl.ds(hb, BATCH // 16)], hxu_v)
    pltpu.sync_copy(xi_hbm.at[pl.ds(hb, BATCH // 16)], hxi_v)
    zrow16 = lanes * 0

    @pl.when(sid == 0)
    def _():
        def zsh(c, _2):
            hu_v[0, pl.ds(c * 16, 16)] = zero16
            return 0

        lax.fori_loop(0, 2 * V // 16, zsh, 0)
        pltpu.sync_copy(hu_v, sh_hist)

    zidx_v[...] = zrow16

    def hz(c, _):
        hu_v[0, pl.ds(c * 16, 16)] = zero16
        return 0

    lax.fori_loop(0, 2 * V // 16, hz, 0)

    def hadd_u(c, _):
        plsc.addupdate_scatter(hu_v, [zrow16, hxu_v[pl.ds(c * 16, 16)]], ones16)
        return 0

    def hadd_i(c, _):
        plsc.addupdate_scatter(hu_v, [zrow16, hxi_v[pl.ds(c * 16, 16)] + V],
                               ones16)
        return 0

    lax.fori_loop(0, BATCH // 256, hadd_u, 0)
    lax.fori_loop(0, BATCH // 256, hadd_i, 0)
    plsc.subcore_barrier()
    pltpu.sync_copy(hu_v, sh_hist.at[zidx_v.at[pl.ds(0, 1)]], add=True)
    plsc.subcore_barrier()

    @pl.when(sid == 0)
    def _():
        pltpu.sync_copy(sh_hist.at[0, pl.ds(0, V)], occu_v)
        pltpu.sync_copy(sh_hist.at[0, pl.ds(V, V)], occi_v)

        def rbody(c, carry):
            oc = (occu_v[pl.ds(c * 16, 16)] + occi_v[pl.ds(c * 16, 16)])
            occ_v[pl.ds(c * 16, 16)] = oc
            pr = jnp.where(oc > 0.0, 1.0, 0.0).astype(f32)
            cs = plsc.cumsum(pr)
            rank_v[pl.ds(c * 16, 16)] = (cs - pr + carry).astype(i32)
            return carry + cs[15]

        lax.fori_loop(0, V // 16, rbody, jnp.float32(0.0))
        pltpu.sync_copy(rank_v, sh_rank)

    plsc.subcore_barrier()

    r0 = sid * RPT
    pltpu.sync_copy(sh_rank.at[pl.ds(r0, RPT)], myrank_v)
    pltpu.sync_copy(entp_hbm.at[pl.ds(r0, RPT)], ent_blk)
    pltpu.sync_copy(pk_hbm.at[pl.ds(r0 * 8, RPT * 8)], pkb_v)
    bmask = jnp.where(lanes < NVS, 1.0, 0.0).astype(f32)

    for half in range(2):
        hoff = half * (RPT // 2)
        pltpu.async_copy(n_hbm.at[myrank_v.at[pl.ds(hoff, RPT // 2)]],
                         n_blk, sem).wait()

        def rowbody(rl, _, hoff=hoff):
            r = hoff + rl
            for c in range(16):
                ca = 32 * c
                cb = 32 * c + 16
                a = (ent_blk[r, pl.ds(ca % 128, 16)]
                     + ent_blk[r, pl.ds(128 + ca % 128, 16)]
                     * n_blk[rl, pl.ds(ca, 16)])
                b = (ent_blk[r, pl.ds(cb % 128, 16)]
                     + ent_blk[r, pl.ds(128 + cb % 128, 16)]
                     * n_blk[rl, pl.ds(cb, 16)])
                w = plsc.bitcast(
                    plsc.pack(a, b, format=plsc.PackFormat.INTERLEAVED), i32)
                mrow_v[r, pl.ds(16 * c, 16)] = w
            return 0

        lax.fori_loop(0, RPT // 2, rowbody, 0)

        def quadbody(q, _, hoff=hoff):
            qb = hoff + q * 16
            lc16 = plsc.load_gather(pkb_v, [(lanes + qb) * 8 + 1])
            sc16 = plsc.load_gather(pkb_v, [(lanes + qb) * 8 + 2])
            bs16 = jnp.zeros((16,), f32)
            for rl in range(16):
                nb = n_blk[q * 16 + rl, pl.ds(ECOLS, 16)] * bmask
                sb = plsc.cumsum(nb)[15]
                bs16 = bs16 + jnp.where(lanes == rl,
                                        jnp.full((16,), sb, f32), 0.0)
            bs64_v[pl.ds(qb, 16)] = 4.0 * lc16 + sc16 * bs16
            return 0

        lax.fori_loop(0, RPT // 32, quadbody, 0)

    @pl.when(cid == 0)
    def _():
        pltpu.sync_copy(mrow_v, tab0_hbm.at[pl.ds(r0, RPT)])

    @pl.when(cid == 1)
    def _():
        pltpu.sync_copy(mrow_v, tab1_hbm.at[pl.ds(r0, RPT)])

    pltpu.sync_copy(bs64_v, sh_bsum.at[pl.ds(r0, RPT)])

    @pl.when(sid == 0)
    def _():
        pltpu.sync_copy(pk_hbm, pkkl_v)

        def zbody(c, _2):
            cnt_v[pl.ds(c * 16, 16)] = zero16
            uniq_v[pl.ds(c * 16, 16)] = zero16
            return 0

        lax.fori_loop(0, V // 16, zbody, 0)

        def sbody(c, _2):
            oc = occ_v[pl.ds(c * 16, 16)]
            rk = rank_v[pl.ds(c * 16, 16)]
            pr = oc > 0.0
            plsc.store_scatter(cnt_v, [rk], oc, mask=pr)
            ids16 = (lanes + c * 16).astype(f32)
            plsc.store_scatter(uniq_v, [rk], ids16, mask=pr)
            return 0

        lax.fori_loop(0, V // 16, sbody, 0)

        def gbody(c, _2):
            up = uniq_v[pl.ds(c * 16, 16)].astype(i32)
            r2_v[pl.ds(c * 16, 16)] = plsc.load_gather(cnt_v, [up])
            return 0

        lax.fori_loop(0, V // 16, gbody, 0)

        def nbody(c, accs):
            au, ai = accs
            r2c = r2_v[pl.ds(c * 16, 16)]
            ou = occu_v[pl.ds(c * 16, 16)]
            oi = occi_v[pl.ds(c * 16, 16)]
            rr = _rcp(r2c)
            au = au + jnp.where(ou > 0.0, ou * rr, 0.0)
            ai = ai + jnp.where(oi > 0.0, oi * rr, 0.0)
            return (au, ai)

        au, ai = lax.fori_loop(0, V // 16, nbody, (jnp.zeros((16,), f32),
                                                   jnp.zeros((16,), f32)))
        run16 = _rcp(jnp.full((16,), _lanesum(au), f32)) * jnp.float32(N_USERS)
        rin16 = _rcp(jnp.full((16,), _lanesum(ai), f32)) * jnp.float32(M_ITEMS)

        def kbody(c, acc):
            oc = occ_v[pl.ds(c * 16, 16)]
            cp = cnt_v[pl.ds(c * 16, 16)]
            kvc = plsc.load_gather(pkkl_v, [(lanes + c * 16) * 8])
            ids16 = (lanes + c * 16).astype(f32)
            sel = (jnp.where(ids16 <= jnp.float32(N_USERS), run16, 0.0)
                   + jnp.where(ids16 > jnp.float32(N_USERS), rin16, 0.0))
            term = jnp.where(oc > 0.0, oc * _rcp(cp), 0.0)
            return acc + kvc * term * sel

        kacc = lax.fori_loop(0, V // 16, kbody, jnp.zeros((16,), f32))
        krs = _lanesum(kacc)
        kbuf[...] = jnp.where(lanes == 0, jnp.full((16,), krs, f32), 0.0)

        @pl.when(cid == 0)
        def _():
            pltpu.sync_copy(kbuf, krs_hbm)

    plsc.subcore_barrier()

    pltpu.sync_copy(sh_bsum, bsum_v)
    wid = sid * 2 + cid
    base = wid * BPW
    pltpu.sync_copy(xu_hbm.at[pl.ds(base, BPW)], idxu_v)
    pltpu.sync_copy(xi_hbm.at[pl.ds(base, BPW)], idxi_v)
    lanes16 = lanes * 16

    def phase2(tab):
        def fire(g, ru, ri, su, si):
            pltpu.async_copy(tab.at[idxu_v.at[pl.ds(g * GROUP, GROUP)]], ru, su)
            pltpu.async_copy(tab.at[idxi_v.at[pl.ds(g * GROUP, GROUP)]], ri, si)

        def wait(g, ru, ri, su, si):
            pltpu.make_async_copy(tab.at[idxu_v.at[pl.ds(g * GROUP, GROUP)]],
                                  ru, su).wait()
            pltpu.make_async_copy(tab.at[idxi_v.at[pl.ds(g * GROUP, GROUP)]],
                                  ri, si).wait()

        def compute(g, ru, ri):
            def group16(g2, _):
                goff = g2 * 16

                def elem(le, _2):
                    e = goff + le
                    acc = jnp.zeros((16,), f32)
                    for c in range(TABW // 16):
                        ui = ru[e, pl.ds(c * 16, 16)]
                        wi = ri[e, pl.ds(c * 16, 16)]
                        u32 = plsc.bitcast(ui, jnp.bfloat16)
                        w32 = plsc.bitcast(wi, jnp.bfloat16)
                        p32 = u32 * w32
                        pa, pb = plsc.unpack(
                            p32, format=plsc.PackFormat.INTERLEAVED,
                            preferred_element_type=f32)
                        acc = acc + pa + pb
                    red_v[pl.ds(le * 16, 16)] = acc
                    return 0

                lax.fori_loop(0, 16, elem, 0)
                s16 = jnp.zeros((16,), f32)
                for c in range(16):
                    s16 = s16 + plsc.load_gather(red_v, [lanes16 + c])
                boff = g * GROUP + goff
                bu = plsc.load_gather(bsum_v, [idxu_v[pl.ds(boff, 16)]])
                bi = plsc.load_gather(bsum_v, [idxi_v[pl.ds(boff, 16)]])
                s_v[pl.ds(boff, 16)] = (s16 + bu + bi) * f32(1.0 / NVS)
                return 0

            lax.fori_loop(0, GROUP // 16, group16, 0)

        fire(0, ru0, ri0, semu0, semi0)

        def hbody(h, _):
            g0 = 2 * h
            wait(g0, ru0, ri0, semu0, semi0)
            fire(g0 + 1, ru1, ri1, semu1, semi1)
            compute(g0, ru0, ri0)
            wait(g0 + 1, ru1, ri1, semu1, semi1)

            @pl.when(g0 + 2 < NG)
            def _():
                fire(g0 + 2, ru0, ri0, semu0, semi0)

            compute(g0 + 1, ru1, ri1)
            return 0

        lax.fori_loop(0, NG // 2, hbody, 0)

    @pl.when(cid == 0)
    def _():
        phase2(tab0_hbm)

    @pl.when(cid == 1)
    def _():
        phase2(tab1_hbm)

    pltpu.sync_copy(s_v, s_hbm.at[pl.ds(base, BPW)])


def _sc_main(n_pad, entp, pk1, xu, xi):
    f32 = jnp.float32
    i32 = jnp.int32
    mesh = plsc.VectorSubcoreMesh(core_axis_name="c", subcore_axis_name="s")
    k = functools.partial(
        pl.kernel,
        out_type=[
            jax.ShapeDtypeStruct((BATCH,), f32),
            jax.ShapeDtypeStruct((16,), f32),
            jax.ShapeDtypeStruct((V, TABW), i32),
            jax.ShapeDtypeStruct((V, TABW), i32),
        ],
        mesh=mesh,
        compiler_params=pltpu.CompilerParams(needs_layout_passes=False),
        scratch_types=[
            pltpu.VMEM_SHARED((V,), i32),
            pltpu.VMEM_SHARED((V,), f32),
            pltpu.VMEM_SHARED((1, 2 * V), f32),
            pltpu.VMEM((1, 2 * V), f32),
            pltpu.VMEM((16,), jnp.int32),
            pltpu.VMEM((BATCH // 16,), jnp.int32),
            pltpu.VMEM((BATCH // 16,), jnp.int32),
            pltpu.VMEM((V,), f32),
            pltpu.VMEM((V,), f32),
            pltpu.VMEM((V,), f32),
            pltpu.VMEM((V,), i32),
            pltpu.VMEM((V,), f32),
            pltpu.VMEM((V,), f32),
            pltpu.VMEM((V,), f32),
            pltpu.VMEM((16,), f32),
            pltpu.VMEM((8 * V,), f32),
            pltpu.VMEM((RPT,), i32),
            pltpu.VMEM((RPT // 2, NPC), f32),
            pltpu.VMEM((RPT, 256), f32),
            pltpu.VMEM((RPT * 8,), f32),
            pltpu.VMEM((RPT, TABW), i32),
            pltpu.VMEM((RPT,), f32),
            pltpu.VMEM((BPW,), i32),
            pltpu.VMEM((BPW,), i32),
            pltpu.VMEM((V,), f32),
            pltpu.VMEM((GROUP, TABW), i32),
            pltpu.VMEM((GROUP, TABW), i32),
            pltpu.VMEM((GROUP, TABW), i32),
            pltpu.VMEM((GROUP, TABW), i32),
            pltpu.VMEM((256,), f32),
            pltpu.VMEM((BPW,), f32),
            pltpu.SemaphoreType.DMA,
            pltpu.SemaphoreType.DMA,
            pltpu.SemaphoreType.DMA,
            pltpu.SemaphoreType.DMA,
            pltpu.SemaphoreType.DMA,
        ],
    )(_sc_main_body)
    return k(n_pad, entp, pk1, xu, xi)


@jax.jit
def _run(x, alpha, gbm, gbs, bias_weight, entity_weight, n_pad, noise_g):
    f32 = jnp.float32
    xu = x[:, 0]
    xi = x[:, 1]
    bias_pad = jnp.concatenate(
        [bias_weight, jnp.tile(jnp.array([[0.0, 1.0]], f32), (V - VOCAB, 1))], axis=0)
    ent_pad = jnp.concatenate(
        [entity_weight,
         jnp.concatenate([jnp.zeros((V - VOCAB, EMBED), f32),
                          jnp.ones((V - VOCAB, EMBED), f32)], axis=1)], axis=0)

    pk, entp, klg, std = pl.pallas_call(
        _stage_a,
        out_shape=[
            jax.ShapeDtypeStruct((V, 8), f32),
            jax.ShapeDtypeStruct((V, 2 * EMBED), f32),
            jax.ShapeDtypeStruct((1, 1), f32),
            jax.ShapeDtypeStruct((1, 1), f32),
        ],
    )(bias_pad, ent_pad, gbm.reshape(1, 1), gbs.reshape(1, 1),
      alpha.reshape(1, 1))

    s, krs, _, _ = _sc_main(n_pad, entp, pk.reshape(8 * V), xu, xi)

    gb = gbm + jnp.abs(gbs) * noise_g
    pred = gb + s.reshape(1, BATCH)
    klt = klg.reshape(1) + krs[0:1]
    return pred, std.reshape(1), klt


def kernel(x, alpha, global_bias_mean, global_bias_scale, bias_weight, entity_weight):
    return _run(x, alpha, global_bias_mean, global_bias_scale,
                bias_weight, entity_weight, jnp.asarray(_N_PAD),
                jnp.asarray(_NOISE_G))
